# baseline (speedup 1.0000x reference)
import functools

import jax
import jax.numpy as jnp
from jax import lax
from jax.experimental import pallas as pl
from jax.experimental.pallas import tpu as pltpu
from jax.experimental.pallas import tpu_sc as plsc

B, T, K, D = 1024, 50, 8, 128
BT = B * T
L = 16
C = 16
CK = C * K
NIDX = 2 * C + 3 * CK


def _proj_body(a_ref, w_ref, b_ref, o_ref, *, zero_first: bool, block_rows: int):
    a = a_ref[...]
    if zero_first:
        row = lax.broadcasted_iota(jnp.int32, a.shape, 0) + pl.program_id(0) * block_rows
        a = jnp.where(row == 0, 0.0, a)
    o_ref[...] = jnp.dot(a, w_ref[...], preferred_element_type=jnp.float32) + b_ref[...]


def _project(table, wblk, bias, zero_first):
    n = table.shape[0]
    r = 2000 if n % 2000 == 0 else n
    grid = n // r
    return pl.pallas_call(
        functools.partial(_proj_body, zero_first=zero_first, block_rows=r),
        grid=(grid,),
        in_specs=[
            pl.BlockSpec((r, D), lambda i: (i, 0)),
            pl.BlockSpec((D, D), lambda i: (0, 0)),
            pl.BlockSpec((1, D), lambda i: (0, 0)),
        ],
        out_specs=pl.BlockSpec((r, D), lambda i: (i, 0)),
        out_shape=jax.ShapeDtypeStruct((n, D), jnp.float32),
    )(table, wblk, bias)


def _cnt_body(a_ref, t_ref, c_ref, g_ref, p_ref, o_ref, k_ref):
    g = g_ref[...]
    p = p_ref[...]
    for f, x_ref in enumerate((a_ref, t_ref, c_ref)):
        x = x_ref[...]
        m = (x != 0).astype(jnp.float32)
        cnt = jnp.dot(m, g, preferred_element_type=jnp.float32)
        o_ref[:, f, :] = cnt.astype(jnp.int32)
        perm = jnp.dot(x.astype(jnp.float32), p,
                       preferred_element_type=jnp.float32,
                       precision=lax.Precision.HIGHEST)
        k_ref[:, f, :] = (perm + 0.5).astype(jnp.int32)


def _cnt_chunks(actors, themes, constraints):
    nchunks = BT // C
    rc = 400
    grid = nchunks // rc
    i = jnp.arange(CK)[:, None]
    j = jnp.arange(2 * L)[None, :]
    gmat = (i // K == j % C).astype(jnp.float32)
    jj = jnp.arange(CK)[None, :]
    pmat = (jj == (i % K) * C + i // K).astype(jnp.float32)
    spec_in = pl.BlockSpec((rc, CK), lambda i: (i, 0))
    return pl.pallas_call(
        _cnt_body,
        grid=(grid,),
        in_specs=[spec_in, spec_in, spec_in,
                  pl.BlockSpec((CK, 2 * L), lambda i: (0, 0)),
                  pl.BlockSpec((CK, CK), lambda i: (0, 0))],
        out_specs=[pl.BlockSpec((rc, 3, 2 * L), lambda i: (i, 0, 0)),
                   pl.BlockSpec((rc, 3, CK), lambda i: (i, 0, 0))],
        out_shape=[jax.ShapeDtypeStruct((nchunks, 3, 2 * L), jnp.int32),
                   jax.ShapeDtypeStruct((nchunks, 3, CK), jnp.int32)],
    )(actors.reshape(nchunks, CK), themes.reshape(nchunks, CK),
      constraints.reshape(nchunks, CK), gmat, pmat)


def _make_sc_encode(nc, ns):
    nw = nc * ns
    cpw = BT // nw
    nchunk = cpw // C

    mesh = plsc.VectorSubcoreMesh(core_axis_name="c", subcore_axis_name="s")

    @functools.partial(
        pl.kernel,
        mesh=mesh,
        out_type=jax.ShapeDtypeStruct((BT, D), jnp.float32),
        scratch_types=[
            pltpu.VMEM((NIDX,), jnp.int32),
            pltpu.VMEM((NIDX,), jnp.int32),
            pltpu.VMEM((4, C, D), jnp.float32),
            pltpu.VMEM((4, C, D), jnp.float32),
            pltpu.VMEM((3, 2 * L), jnp.int32),
            pltpu.VMEM((3, 2 * L), jnp.int32),
            pltpu.VMEM((9, L), jnp.float32),
            pltpu.VMEM((C, D), jnp.float32),
            pltpu.SemaphoreType.DMA,
            pltpu.SemaphoreType.DMA,
            pltpu.SemaphoreType.DMA,
            pltpu.SemaphoreType.DMA,
            pltpu.SemaphoreType.DMA,
        ],
    )
    def sc_encode(pet, pac, pa, pth, pco, ipack, cnts, invtab, zhbm, out_hbm,
                  ipk0, ipk1, acc0, acc1, cnt0, cnt1, invt, ob,
                  sem0, sem1, zsem0, zsem1, osem):
        wid = lax.axis_index("s") * nc + lax.axis_index("c")
        pltpu.sync_copy(invtab, invt)

        ipks = (ipk0, ipk1)
        accs = (acc0, acc1)
        cntb = (cnt0, cnt1)
        sems = (sem0, sem1)
        zsems = (zsem0, zsem1)

        def streams(slot):
            ipk, acc = ipks[slot], accs[slot]
            sts = [(ipk.at[pl.ds(0, C)], acc.at[0]),
                   (ipk.at[pl.ds(C, C)], acc.at[0])]
            tabs = (pa, pth, pco)
            for f in range(3):
                off = 2 * C + f * CK
                for k in range(K):
                    sts.append((ipk.at[pl.ds(off + k * C, C)], acc.at[1 + f]))
            return [(pet, sts[0]), (pac, sts[1])] + [
                (tabs[f], sts[2 + f * K + k]) for f in range(3) for k in range(K)]

        def issue(cid, slot):
            ipk, cnt, sem = ipks[slot], cntb[slot], sems[slot]
            pltpu.sync_copy(ipack.at[cid], ipk)
            pltpu.make_async_copy(zhbm, accs[slot], zsems[slot]).wait()
            for tab, (idx, dst) in streams(slot):
                pltpu.async_copy(tab.at[idx], dst, sem, add=True)
            pltpu.async_copy(cnts.at[cid], cnt, sem)

        def drain(cid, slot):
            cnt, sem = cntb[slot], sems[slot]
            for tab, (idx, dst) in streams(slot):
                pltpu.make_async_copy(tab.at[idx], dst, sem).wait()
            pltpu.make_async_copy(cnts.at[cid], cnt, sem).wait()

        def compute(cid, slot):
            acc, cnt = accs[slot], cntb[slot]
            drain(cid, slot)

            def tok(t, c):
                ivs = []
                for f in range(3):
                    cv = cnt[f, pl.ds(t, L)]
                    ivs.append(invt[cv[0], :])
                for d in range(D // L):
                    sl = pl.ds(d * L, L)
                    v = acc[0, t, sl]
                    for f in range(3):
                        v = v + acc[1 + f, t, sl] * ivs[f]
                    ob[t, sl] = v
                return c

            lax.fori_loop(0, C, tok, 0)
            pltpu.async_copy(zhbm, acc, zsems[slot])

        def store(cid):
            pltpu.async_copy(ob, out_hbm.at[pl.ds(cid * C, C)], osem)

        def wait_store(cid):
            pltpu.make_async_copy(ob, out_hbm.at[pl.ds(cid * C, C)], osem).wait()

        pltpu.async_copy(zhbm, acc0, zsem0)
        pltpu.async_copy(zhbm, acc1, zsem1)

        base = wid * nchunk
        issue(base, 0)

        def outer(i, carry):
            for b in range(2):
                g = i * 2 + b
                cid = base + g

                @pl.when(g + 1 < nchunk)
                def _():
                    issue(cid + 1, 1 - b)

                @pl.when(g > 0)
                def _():
                    wait_store(cid)
                compute(cid, b)
                store(cid)
            return carry

        lax.fori_loop(0, nchunk // 2, outer, 0)
        wait_store(base)
        pltpu.make_async_copy(zhbm, acc0, zsem0).wait()
        pltpu.make_async_copy(zhbm, acc1, zsem1).wait()

    return sc_encode


def kernel(event_type, action, actors, themes, constraints,
           event_type_emb, action_emb, actor_emb, theme_emb, constraint_emb,
           W, b):
    wr = W.reshape(5, D, D)
    zero_bias = jnp.zeros((1, D), jnp.float32)
    pet = _project(event_type_emb, wr[0], b.reshape(1, D), False)
    pac = _project(action_emb, wr[1], zero_bias, False)
    pa = _project(actor_emb, wr[2], zero_bias, True)
    pth = _project(theme_emb, wr[3], zero_bias, True)
    pco = _project(constraint_emb, wr[4], zero_bias, True)

    info = plsc.get_sparse_core_info()
    nchunks = BT // C

    cnts, kidx = _cnt_chunks(actors, themes, constraints)

    ipack = jnp.concatenate(
        [event_type.reshape(nchunks, C), action.reshape(nchunks, C),
         kidx.reshape(nchunks, 3 * CK)], axis=1)

    invtab = jnp.broadcast_to(
        (1.0 / jnp.maximum(jnp.arange(9, dtype=jnp.float32), 1.0))[:, None], (9, L))

    zhbm = jnp.zeros((4, C, D), jnp.float32)

    sc_encode = _make_sc_encode(info.num_cores, info.num_subcores)
    out = sc_encode(pet, pac, pa, pth, pco, ipack, cnts, invtab, zhbm)
    return out.reshape(B, T, D)


# speedup vs baseline: 1.1578x; 1.1578x over previous
import functools

import jax
import jax.numpy as jnp
from jax import lax
from jax.experimental import pallas as pl
from jax.experimental.pallas import tpu as pltpu
from jax.experimental.pallas import tpu_sc as plsc

B, T, K, D = 1024, 50, 8, 128
BT = B * T
L = 16
C = 80
CK = C * K
CW = C + L
NIDX = 2 * C + 3 * CK


def _proj_body(a_ref, w_ref, b_ref, o_ref, *, zero_first: bool, block_rows: int):
    a = a_ref[...]
    if zero_first:
        row = lax.broadcasted_iota(jnp.int32, a.shape, 0) + pl.program_id(0) * block_rows
        a = jnp.where(row == 0, 0.0, a)
    o_ref[...] = jnp.dot(a, w_ref[...], preferred_element_type=jnp.float32) + b_ref[...]


def _project(table, wblk, bias, zero_first):
    n = table.shape[0]
    r = 2000 if n % 2000 == 0 else n
    grid = n // r
    return pl.pallas_call(
        functools.partial(_proj_body, zero_first=zero_first, block_rows=r),
        grid=(grid,),
        in_specs=[
            pl.BlockSpec((r, D), lambda i: (i, 0)),
            pl.BlockSpec((D, D), lambda i: (0, 0)),
            pl.BlockSpec((1, D), lambda i: (0, 0)),
        ],
        out_specs=pl.BlockSpec((r, D), lambda i: (i, 0)),
        out_shape=jax.ShapeDtypeStruct((n, D), jnp.float32),
    )(table, wblk, bias)


def _cnt_body(a_ref, t_ref, c_ref, g_ref, p_ref, o_ref, k_ref):
    g = g_ref[...]
    p = p_ref[...]
    for f, x_ref in enumerate((a_ref, t_ref, c_ref)):
        x = x_ref[...]
        m = (x != 0).astype(jnp.float32)
        cnt = jnp.dot(m, g, preferred_element_type=jnp.float32)
        o_ref[:, f, :] = cnt.astype(jnp.int32)
        perm = jnp.dot(x.astype(jnp.float32), p,
                       preferred_element_type=jnp.float32,
                       precision=lax.Precision.HIGHEST)
        k_ref[:, f, :] = (perm + 0.5).astype(jnp.int32)


def _cnt_chunks(actors, themes, constraints):
    nchunks = BT // C
    rc = next(r for r in (400, 320, 200, 160, 128, 100, 80, 64, 40, 32)
              if nchunks % r == 0)
    grid = nchunks // rc
    i = jnp.arange(CK)[:, None]
    j = jnp.arange(CW)[None, :]
    gmat = (i // K == j % C).astype(jnp.float32)
    jj = jnp.arange(CK)[None, :]
    pmat = (jj == (i % K) * C + i // K).astype(jnp.float32)
    spec_in = pl.BlockSpec((rc, CK), lambda i: (i, 0))
    return pl.pallas_call(
        _cnt_body,
        grid=(grid,),
        in_specs=[spec_in, spec_in, spec_in,
                  pl.BlockSpec((CK, CW), lambda i: (0, 0)),
                  pl.BlockSpec((CK, CK), lambda i: (0, 0))],
        out_specs=[pl.BlockSpec((rc, 3, CW), lambda i: (i, 0, 0)),
                   pl.BlockSpec((rc, 3, CK), lambda i: (i, 0, 0))],
        out_shape=[jax.ShapeDtypeStruct((nchunks, 3, CW), jnp.int32),
                   jax.ShapeDtypeStruct((nchunks, 3, CK), jnp.int32)],
    )(actors.reshape(nchunks, CK), themes.reshape(nchunks, CK),
      constraints.reshape(nchunks, CK), gmat, pmat)


def _make_sc_encode(nc, ns):
    nw = nc * ns
    cpw = BT // nw
    nchunk = cpw // C

    mesh = plsc.VectorSubcoreMesh(core_axis_name="c", subcore_axis_name="s")

    @functools.partial(
        pl.kernel,
        mesh=mesh,
        out_type=jax.ShapeDtypeStruct((BT, D), jnp.float32),
        scratch_types=[
            pltpu.VMEM((NIDX,), jnp.int32),
            pltpu.VMEM((NIDX,), jnp.int32),
            pltpu.VMEM((4, C, D), jnp.float32),
            pltpu.VMEM((4, C, D), jnp.float32),
            pltpu.VMEM((3, CW), jnp.int32),
            pltpu.VMEM((3, CW), jnp.int32),
            pltpu.VMEM((9, L), jnp.float32),
            pltpu.VMEM((C, D), jnp.float32),
            pltpu.SemaphoreType.DMA,
            pltpu.SemaphoreType.DMA,
            pltpu.SemaphoreType.DMA,
            pltpu.SemaphoreType.DMA,
            pltpu.SemaphoreType.DMA,
        ],
    )
    def sc_encode(pet, pac, pa, pth, pco, ipack, cnts, invtab, zhbm, out_hbm,
                  ipk0, ipk1, acc0, acc1, cnt0, cnt1, invt, ob,
                  sem0, sem1, zsem0, zsem1, osem):
        wid = lax.axis_index("s") * nc + lax.axis_index("c")
        pltpu.sync_copy(invtab, invt)

        ipks = (ipk0, ipk1)
        accs = (acc0, acc1)
        cntb = (cnt0, cnt1)
        sems = (sem0, sem1)
        zsems = (zsem0, zsem1)

        def streams(slot):
            ipk, acc = ipks[slot], accs[slot]
            sts = [(ipk.at[pl.ds(0, C)], acc.at[0]),
                   (ipk.at[pl.ds(C, C)], acc.at[0])]
            tabs = (pa, pth, pco)
            for f in range(3):
                off = 2 * C + f * CK
                for k in range(K):
                    sts.append((ipk.at[pl.ds(off + k * C, C)], acc.at[1 + f]))
            return [(pet, sts[0]), (pac, sts[1])] + [
                (tabs[f], sts[2 + f * K + k]) for f in range(3) for k in range(K)]

        def issue(cid, slot):
            ipk, cnt, sem = ipks[slot], cntb[slot], sems[slot]
            pltpu.sync_copy(ipack.at[cid], ipk)
            pltpu.make_async_copy(zhbm, accs[slot], zsems[slot]).wait()
            for tab, (idx, dst) in streams(slot):
                pltpu.async_copy(tab.at[idx], dst, sem, add=True)
            pltpu.async_copy(cnts.at[cid], cnt, sem)

        def drain(cid, slot):
            cnt, sem = cntb[slot], sems[slot]
            for tab, (idx, dst) in streams(slot):
                pltpu.make_async_copy(tab.at[idx], dst, sem).wait()
            pltpu.make_async_copy(cnts.at[cid], cnt, sem).wait()

        def compute(cid, slot):
            acc, cnt = accs[slot], cntb[slot]
            drain(cid, slot)

            def tok(t, c):
                ivs = []
                for f in range(3):
                    cv = cnt[f, pl.ds(t, L)]
                    ivs.append(invt[cv[0], :])
                for d in range(D // L):
                    sl = pl.ds(d * L, L)
                    v = acc[0, t, sl]
                    for f in range(3):
                        v = v + acc[1 + f, t, sl] * ivs[f]
                    ob[t, sl] = v
                return c

            lax.fori_loop(0, C, tok, 0)
            pltpu.async_copy(zhbm, acc, zsems[slot])

        def store(cid):
            pltpu.async_copy(ob, out_hbm.at[pl.ds(cid * C, C)], osem)

        def wait_store(cid):
            pltpu.make_async_copy(ob, out_hbm.at[pl.ds(cid * C, C)], osem).wait()

        pltpu.async_copy(zhbm, acc0, zsem0)
        pltpu.async_copy(zhbm, acc1, zsem1)

        base = wid * nchunk
        issue(base, 0)

        def outer(i, carry):
            for b in range(2):
                g = i * 2 + b
                cid = base + g

                @pl.when(g + 1 < nchunk)
                def _():
                    issue(cid + 1, 1 - b)

                @pl.when(g > 0)
                def _():
                    wait_store(cid)
                compute(cid, b)
                store(cid)
            return carry

        lax.fori_loop(0, nchunk // 2, outer, 0)
        wait_store(base)
        pltpu.make_async_copy(zhbm, acc0, zsem0).wait()
        pltpu.make_async_copy(zhbm, acc1, zsem1).wait()

    return sc_encode


def kernel(event_type, action, actors, themes, constraints,
           event_type_emb, action_emb, actor_emb, theme_emb, constraint_emb,
           W, b):
    wr = W.reshape(5, D, D)
    zero_bias = jnp.zeros((1, D), jnp.float32)
    pet = _project(event_type_emb, wr[0], b.reshape(1, D), False)
    pac = _project(action_emb, wr[1], zero_bias, False)
    pa = _project(actor_emb, wr[2], zero_bias, True)
    pth = _project(theme_emb, wr[3], zero_bias, True)
    pco = _project(constraint_emb, wr[4], zero_bias, True)

    info = plsc.get_sparse_core_info()
    nchunks = BT // C

    cnts, kidx = _cnt_chunks(actors, themes, constraints)

    ipack = jnp.concatenate(
        [event_type.reshape(nchunks, C), action.reshape(nchunks, C),
         kidx.reshape(nchunks, 3 * CK)], axis=1)

    invtab = jnp.broadcast_to(
        (1.0 / jnp.maximum(jnp.arange(9, dtype=jnp.float32), 1.0))[:, None], (9, L))

    zhbm = jnp.zeros((4, C, D), jnp.float32)

    sc_encode = _make_sc_encode(info.num_cores, info.num_subcores)
    out = sc_encode(pet, pac, pa, pth, pco, ipack, cnts, invtab, zhbm)
    return out.reshape(B, T, D)
